# Initial kernel scaffold; baseline (speedup 1.0000x reference)
import jax
import jax.numpy as jnp
from jax.experimental import pallas as pl


def kernel(pos, vel, edge_index, mesh_edge_attr, other_features, node_type, vel_mean, vel_std, t_span, W_e1, b_e1, W_n1, b_n1, W_n2, b_n2):
    raise NotImplementedError("write your pallas kernel here")


# speedup vs baseline: 1.0341x; 1.0341x over previous
import jax
import jax.numpy as jnp
from jax.experimental import pallas as pl


def _node_mlp_kernel(n_in_ref, w1_ref, b1_ref, w2_ref, b2_ref, out_ref):
    h = jnp.maximum(jnp.dot(n_in_ref[...], w1_ref[...],
                            preferred_element_type=jnp.float32) + b1_ref[...], 0.0)
    out_ref[...] = jnp.dot(h, w2_ref[...], preferred_element_type=jnp.float32) + b2_ref[...]


def _node_mlp(n_in, W_n1, b_n1, W_n2, b_n2):
    N = n_in.shape[0]
    b1 = b_n1.reshape(1, -1)
    b2 = jnp.zeros((1, 128), jnp.float32).at[0, :3].set(b_n2)
    W2 = jnp.zeros((128, 128), jnp.float32).at[:, :3].set(W_n2)
    out = pl.pallas_call(
        _node_mlp_kernel,
        out_shape=jax.ShapeDtypeStruct((N, 128), jnp.float32),
    )(n_in, W_n1, b1, W2, b2)
    return out[:, :3]


def kernel(pos, vel, edge_index, mesh_edge_attr, other_features, node_type,
           vel_mean, vel_std, t_span, W_e1, b_e1, W_n1, b_n1, W_n2, b_n2):
    N = pos.shape[0]
    N_TYPES = 9
    T = t_span.shape[0]
    type_onehot = jax.nn.one_hot(node_type[:, 0], N_TYPES, dtype=jnp.float32)
    src = edge_index[0]
    dst = edge_index[1]

    def f(p, v):
        vel_n = (v - vel_mean) / vel_std
        rel_pos = jnp.take(p, src, axis=0) - jnp.take(p, dst, axis=0)
        dist = jnp.sqrt(jnp.sum(rel_pos * rel_pos, axis=-1, keepdims=True) + 1e-8)
        rel_vel = jnp.take(vel_n, src, axis=0) - jnp.take(vel_n, dst, axis=0)
        e_in = jnp.concatenate([rel_pos, dist, rel_vel, mesh_edge_attr], axis=-1)
        m = jax.nn.relu(e_in @ W_e1 + b_e1)
        agg = jax.ops.segment_sum(m, dst, num_segments=N)
        n_in = jnp.concatenate([agg, vel_n, other_features, type_onehot], axis=-1)
        acc = _node_mlp(n_in, W_n1, b_n1, W_n2, b_n2)
        dvel = acc * vel_std
        return v, dvel

    ps = [pos]
    vs = [vel]
    p, v = pos, vel
    for i in range(T - 1):
        dt = t_span[i + 1] - t_span[i]
        k1p, k1v = f(p, v)
        k2p, k2v = f(p + dt * 0.5 * k1p, v + dt * 0.5 * k1v)
        k3p, k3v = f(p + dt * 0.5 * k2p, v + dt * 0.5 * k2v)
        k4p, k4v = f(p + dt * k3p, v + dt * k3v)
        p = p + dt / 6.0 * (k1p + 2.0 * k2p + 2.0 * k3p + k4p)
        v = v + dt / 6.0 * (k1v + 2.0 * k2v + 2.0 * k3v + k4v)
        ps.append(p)
        vs.append(v)
    return jnp.stack(ps, axis=0), jnp.stack(vs, axis=0)


# speedup vs baseline: 1.2367x; 1.1959x over previous
import functools

import jax
import jax.numpy as jnp
from jax import lax
from jax.experimental import pallas as pl
from jax.experimental.pallas import tpu as pltpu
from jax.experimental.pallas import tpu_sc as plsc

_N = 10000
_E = 320000
_H = 128
_C = 64
_NSLOT = _N * _C
_NROW = _NSLOT // 8
_NB = 200
_RB = _NB * 8
_NB_N = 1000
_BIG = 1.0e9


def _sc_gather(table, idx):
    info = plsc.get_sparse_core_info()
    nw = info.num_cores * info.num_subcores
    rows_pw = _NSLOT // nw
    ch = 2000
    nch = rows_pw // ch
    mesh = plsc.VectorSubcoreMesh(core_axis_name="c", subcore_axis_name="s")

    @functools.partial(
        pl.kernel,
        out_type=jax.ShapeDtypeStruct((_NSLOT, 16), jnp.float32),
        mesh=mesh,
        compiler_params=pltpu.CompilerParams(use_tc_tiling_on_sc=False),
        scratch_types=[
            pltpu.VMEM((ch,), jnp.int32),
            pltpu.VMEM((ch, 16), jnp.float32),
            pltpu.SemaphoreType.DMA,
        ],
    )
    def k(table_hbm, idx_hbm, out_hbm, idx_v, rows_v, sem):
        wid = lax.axis_index("s") * info.num_cores + lax.axis_index("c")
        base = wid * rows_pw
        for kk in range(nch):
            off = base + kk * ch
            pltpu.sync_copy(idx_hbm.at[pl.ds(off, ch)], idx_v)
            pltpu.async_copy(table_hbm.at[idx_v], rows_v, sem).wait()
            pltpu.sync_copy(rows_v, out_hbm.at[pl.ds(off, ch)])

    return k(table, idx)


def _edge_body(xs_ref, xt_ref, attr_ref, wbd_ref, bt_ref, p_ref, s_ref,
               wsum_ref, out_ref):
    xs = xs_ref[...]
    xtb = xt_ref[...]
    xt128 = jnp.concatenate([xtb] * 8, axis=1)
    dstx = jnp.broadcast_to(xt128[:, None, :], (_NB, 8, 128)).reshape(_RB, 128)
    g = xs - dstx
    sq = g * g
    d2 = jnp.dot(sq, p_ref[...], preferred_element_type=jnp.float32)
    dist = jnp.sqrt(d2 + 1e-8)
    lane = lax.broadcasted_iota(jnp.int32, (_RB, 128), 1)
    e_in = jnp.where(lane % 16 == 6, dist, g) + attr_ref[...]
    z = jnp.dot(e_in, wbd_ref[...], preferred_element_type=jnp.float32) + bt_ref[...]
    m = jnp.maximum(z, 0.0)
    t = jnp.dot(s_ref[...], m, preferred_element_type=jnp.float32)
    out_ref[...] = jnp.dot(t, wsum_ref[...], preferred_element_type=jnp.float32)


def _tc_edge(xs, xt, attr, wbd, bt, pmat, smat, wsum):
    grid = _N // _NB
    return pl.pallas_call(
        _edge_body,
        grid=(grid,),
        in_specs=[
            pl.BlockSpec((_RB, 128), lambda i: (i, 0)),
            pl.BlockSpec((_NB, 16), lambda i: (i, 0)),
            pl.BlockSpec((_RB, 128), lambda i: (i, 0)),
            pl.BlockSpec((128, 1024), lambda i: (0, 0)),
            pl.BlockSpec((1, 1024), lambda i: (0, 0)),
            pl.BlockSpec((128, 128), lambda i: (0, 0)),
            pl.BlockSpec((_NB, _RB), lambda i: (0, 0)),
            pl.BlockSpec((1024, _H), lambda i: (0, 0)),
        ],
        out_specs=pl.BlockSpec((_NB, _H), lambda i: (i, 0)),
        out_shape=jax.ShapeDtypeStruct((_N, _H), jnp.float32),
    )(xs, xt, attr, wbd, bt, pmat, smat, wsum)


def _node_body(agg_ref, xt_ref, nf_ref, w1_ref, b1_ref, w2_ref, b2_ref, out_ref):
    n_in = jnp.concatenate([agg_ref[...], xt_ref[...], nf_ref[...]], axis=1)
    h = jnp.maximum(
        jnp.dot(n_in, w1_ref[...], preferred_element_type=jnp.float32) + b1_ref[...], 0.0)
    out_ref[...] = jnp.dot(h, w2_ref[...], preferred_element_type=jnp.float32) + b2_ref[...]


def _tc_node(agg, xt, nfeat, w1p, b1, w2p, b2p):
    grid = _N // _NB_N
    return pl.pallas_call(
        _node_body,
        grid=(grid,),
        in_specs=[
            pl.BlockSpec((_NB_N, _H), lambda i: (i, 0)),
            pl.BlockSpec((_NB_N, 16), lambda i: (i, 0)),
            pl.BlockSpec((_NB_N, 32), lambda i: (i, 0)),
            pl.BlockSpec((176, _H), lambda i: (0, 0)),
            pl.BlockSpec((1, _H), lambda i: (0, 0)),
            pl.BlockSpec((_H, 8), lambda i: (0, 0)),
            pl.BlockSpec((1, 8), lambda i: (0, 0)),
        ],
        out_specs=pl.BlockSpec((_NB_N, 8), lambda i: (i, 0)),
        out_shape=jax.ShapeDtypeStruct((_N, 8), jnp.float32),
    )(agg, xt, nfeat, w1p, b1, w2p, b2p)


def kernel(pos, vel, edge_index, mesh_edge_attr, other_features, node_type,
           vel_mean, vel_std, t_span, W_e1, b_e1, W_n1, b_n1, W_n2, b_n2):
    f32 = jnp.float32
    T = t_span.shape[0]
    src = edge_index[0]
    dst = edge_index[1]

    perm = jnp.argsort(dst)
    dst_s = dst[perm]
    src_s = src[perm]
    attr_s = mesh_edge_attr[perm]
    nodes = jnp.arange(_N, dtype=jnp.int32)
    seg_start = jnp.searchsorted(dst_s, nodes).astype(jnp.int32)
    seg_end = jnp.searchsorted(dst_s, nodes, side="right").astype(jnp.int32)
    deg = seg_end - seg_start
    n_pass = (jnp.max(deg) + _C - 1) // _C

    coff = jnp.arange(_C, dtype=jnp.int32)[None, :]

    def build_slots(p):
        k = p * _C + coff
        epos = seg_start[:, None] + k
        valid = k < deg[:, None]
        epos_c = jnp.clip(epos, 0, _E - 1).reshape(-1)
        ssrc = jnp.where(valid, src_s[epos_c].reshape(_N, _C), _N)
        a = jnp.where(valid[..., None], attr_s[epos_c].reshape(_N, _C, 4), 0.0)
        sattr = jnp.zeros((_N, _C, 16), f32).at[:, :, 8:12].set(a)
        return ssrc.reshape(-1).astype(jnp.int32), sattr.reshape(_NROW, 128)

    ssrc0, sattr0 = build_slots(0)

    grp = jnp.arange(128) // 16
    lane16 = jnp.arange(128) % 16
    w16 = jnp.zeros((16, _H), f32)
    w16 = w16.at[0:3].set(W_e1[0:3])
    w16 = w16.at[3:6].set(W_e1[4:7])
    w16 = w16.at[6].set(W_e1[3])
    w16 = w16.at[7].set(-1.0)
    w16 = w16.at[8:12].set(W_e1[7:11])
    wbd = jnp.where((grp[:, None, None] == jnp.arange(8)[None, :, None]),
                    w16[lane16][:, None, :], 0.0).reshape(128, 1024)
    bt = jnp.tile(b_e1, (8,)).reshape(1, 1024)
    pmat = ((lane16[:, None] < 3) & (lane16[None, :] == 6)
            & (grp[:, None] == grp[None, :])).astype(f32)
    smat = (jnp.arange(_RB)[None, :] // 8 == jnp.arange(_NB)[:, None]).astype(f32)
    wsum = (jnp.arange(1024)[:, None] % 128 == jnp.arange(128)[None, :]).astype(f32)

    w1p = jnp.zeros((176, _H), f32)
    w1p = w1p.at[0:128].set(W_n1[0:128])
    w1p = w1p.at[131:134].set(W_n1[128:131])
    w1p = w1p.at[144:160].set(W_n1[131:147])
    w1p = w1p.at[160:169].set(W_n1[147:156])
    b1 = b_n1.reshape(1, _H)
    w2p = jnp.zeros((_H, 8), f32).at[:, 0:3].set(W_n2)
    b2p = jnp.zeros((1, 8), f32).at[0, 0:3].set(b_n2)

    type_onehot = jax.nn.one_hot(node_type[:, 0], 9, dtype=f32)
    nfeat = jnp.zeros((_N, 32), f32)
    nfeat = nfeat.at[:, 0:16].set(other_features)
    nfeat = nfeat.at[:, 16:25].set(type_onehot)

    sent = jnp.zeros((8, 16), f32).at[0, 7].set(_BIG)

    def f(p, v):
        vel_n = (v - vel_mean) / vel_std
        xt_n = jnp.concatenate([p, vel_n, jnp.zeros((_N, 10), f32)], axis=1)
        xt = jnp.concatenate([xt_n, sent], axis=0)
        xs = _sc_gather(xt, ssrc0).reshape(_NROW, 128)
        agg = _tc_edge(xs, xt_n, sattr0, wbd, bt, pmat, smat, wsum)

        def body(pp, acc_agg):
            ssrc_p, sattr_p = build_slots(pp)
            xs_p = _sc_gather(xt, ssrc_p).reshape(_NROW, 128)
            return acc_agg + _tc_edge(xs_p, xt_n, sattr_p, wbd, bt, pmat, smat, wsum)

        agg = lax.fori_loop(1, n_pass, body, agg)
        acc8 = _tc_node(agg, xt_n, nfeat, w1p, b1, w2p, b2p)
        dvel = acc8[:, 0:3] * vel_std
        return v, dvel

    ps = [pos]
    vs = [vel]
    p, v = pos, vel
    for i in range(T - 1):
        dt = t_span[i + 1] - t_span[i]
        k1p, k1v = f(p, v)
        k2p, k2v = f(p + dt * 0.5 * k1p, v + dt * 0.5 * k1v)
        k3p, k3v = f(p + dt * 0.5 * k2p, v + dt * 0.5 * k2v)
        k4p, k4v = f(p + dt * k3p, v + dt * k3v)
        p = p + dt / 6.0 * (k1p + 2.0 * k2p + 2.0 * k3p + k4p)
        v = v + dt / 6.0 * (k1v + 2.0 * k2v + 2.0 * k3v + k4v)
        ps.append(p)
        vs.append(v)
    return jnp.stack(ps, axis=0), jnp.stack(vs, axis=0)
